# baseline (device time: 13061 ns/iter reference)
import jax
import jax.numpy as jnp
from jax import lax
from jax.experimental import pallas as pl
from jax.experimental.pallas import tpu as pltpu

N_DEV = 8


def _cmpx(v, idx, j, dirmask):
    L = v.shape[0]
    up = (idx & j) == 0
    p = jnp.where(up, pltpu.roll(v, L - j, 0), pltpu.roll(v, j, 0))
    take_min = up == dirmask
    return jnp.where(take_min, jnp.minimum(v, p), jnp.maximum(v, p))


def kernel(x):
    m_per, n = x.shape
    M = N_DEV * m_per

    def body(x_ref, out_ref, gather_ref, merge_ref, send_sems, recv_sems):
        my = lax.axis_index("i")

        barrier_sem = pltpu.get_barrier_semaphore()
        for e in range(1, N_DEV):
            pl.semaphore_signal(
                barrier_sem, inc=1,
                device_id=(my ^ e,),
                device_id_type=pl.DeviceIdType.MESH,
            )
        pl.semaphore_wait(barrier_sem, N_DEV - 1)

        v = x_ref[:, :]
        flip = (my & 1) == 1
        idx = lax.broadcasted_iota(jnp.int32, (m_per, n), 0)
        k = 2
        while k <= m_per:
            j = k // 2
            while j >= 1:
                v = _cmpx(v, idx, j, ((idx & k) == 0) ^ flip)
                j //= 2
            k *= 2
        gather_ref[pl.ds(my * m_per, m_per), :] = v

        sends = {}
        recvs = {}
        for e in range(1, N_DEV):
            send = pltpu.make_async_remote_copy(
                src_ref=gather_ref.at[pl.ds(my * m_per, m_per)],
                dst_ref=gather_ref.at[pl.ds(my * m_per, m_per)],
                send_sem=send_sems.at[e - 1],
                recv_sem=recv_sems.at[e - 1],
                device_id=(my ^ e,),
                device_id_type=pl.DeviceIdType.MESH,
            )
            send.start()
            sends[e] = send
            recvs[e] = pltpu.make_async_remote_copy(
                src_ref=gather_ref.at[pl.ds(my * m_per, m_per)],
                dst_ref=gather_ref.at[pl.ds((my ^ e) * m_per, m_per)],
                send_sem=send_sems.at[e - 1],
                recv_sem=recv_sems.at[e - 1],
                device_id=(my ^ e,),
                device_id_type=pl.DeviceIdType.MESH,
            )

        def merge_pair(origin):
            start = (origin & ~1) * m_per
            w = gather_ref[pl.ds(start, 2 * m_per), :]
            widx = lax.broadcasted_iota(jnp.int32, (2 * m_per, n), 0)
            d_asc = ((origin >> 1) & 1) == 0
            j = m_per
            while j >= 1:
                w = _cmpx(w, widx, j, d_asc)
                j //= 2
            merge_ref[pl.ds(start, 2 * m_per), :] = w

        def merge_512(origin):
            start = (origin & ~3) * m_per
            w = merge_ref[pl.ds(start, 4 * m_per), :]
            widx = lax.broadcasted_iota(jnp.int32, (4 * m_per, n), 0)
            d_asc = ((origin >> 2) & 1) == 0
            j = 2 * m_per
            while j >= 1:
                w = _cmpx(w, widx, j, d_asc)
                j //= 2
            merge_ref[pl.ds(start, 4 * m_per), :] = w

        recvs[1].wait_recv()
        merge_pair(my)
        recvs[2].wait_recv()
        recvs[3].wait_recv()
        merge_pair(my ^ 2)
        merge_512(my)

        recvs[4].wait_recv()
        recvs[5].wait_recv()
        merge_pair(my ^ 4)
        recvs[6].wait_recv()
        recvs[7].wait_recv()
        merge_pair(my ^ 6)
        merge_512(my ^ 4)

        half = (my >> 2) & 1
        mine = merge_ref[pl.ds(half * (M // 2), M // 2), :]
        other = merge_ref[pl.ds((1 - half) * (M // 2), M // 2), :]
        w = jnp.where(half == 0, jnp.minimum(mine, other),
                      jnp.maximum(mine, other))
        merge_ref[pl.ds(half * (M // 2), M // 2), :] = w

        q = my >> 1
        mine = merge_ref[pl.ds(q * (M // 4), M // 4), :]
        other = merge_ref[pl.ds((q ^ 1) * (M // 4), M // 4), :]
        w = jnp.where((q & 1) == 0, jnp.minimum(mine, other),
                      jnp.maximum(mine, other))
        merge_ref[pl.ds(q * (M // 4), M // 4), :] = w

        mine = merge_ref[pl.ds(my * m_per, m_per), :]
        other = merge_ref[pl.ds((my ^ 1) * m_per, m_per), :]
        sl = jnp.where((my & 1) == 0, jnp.minimum(mine, other),
                       jnp.maximum(mine, other))

        j = m_per // 2
        while j >= 1:
            sl = _cmpx(sl, idx, j, True)
            j //= 2
        out_ref[:, :] = sl

        for e in range(1, N_DEV):
            sends[e].wait_send()

    return pl.pallas_call(
        body,
        out_shape=jax.ShapeDtypeStruct((m_per, n), x.dtype),
        in_specs=[pl.BlockSpec(memory_space=pltpu.VMEM)],
        out_specs=pl.BlockSpec(memory_space=pltpu.VMEM),
        scratch_shapes=[
            pltpu.VMEM((M, n), x.dtype),
            pltpu.VMEM((M, n), x.dtype),
            pltpu.SemaphoreType.DMA((N_DEV - 1,)),
            pltpu.SemaphoreType.DMA((N_DEV - 1,)),
        ],
        compiler_params=pltpu.CompilerParams(collective_id=0),
    )(x)
